# baseline (speedup 1.0000x reference)
import jax
import jax.numpy as jnp
from jax import lax
from jax.experimental import pallas as pl
from jax.experimental.pallas import tpu as pltpu
from jax.experimental.pallas import tpu_sc as plsc

N0 = 10000
NP = 10240
E0 = 160000
EP = 163840
D = 256
H = 128
B = 64
NC = 2
NS = 16
NW = NC * NS
HALF = NP // NC
TROWS = HALF + 8
DUMMY = HALF
GT = 72
NBLK = 1024
EBLK = 512
CH2 = 64
CH4 = 32
SL = 4
F32 = jnp.float32


def _silu(x):
    return x * jax.nn.sigmoid(x)


def _elu(x):
    return jnp.where(x > 0, x, jnp.exp(jnp.minimum(x, 0.0)) - 1.0)


def _dot(a, b):
    return jnp.dot(a, b, preferred_element_type=F32)


BF = jnp.bfloat16


def _bdot(a, b):
    return jnp.dot(a.astype(BF), b, preferred_element_type=F32)


def _k1_body(h_ref, w1r_ref, w1c_ref, be1_ref,
             prl_ref, prr_ref, pcl_ref, pcr_ref):
    h = h_ref[...]
    pr = _bdot(h, w1r_ref[...]) + be1_ref[...]
    pc = _bdot(h, w1c_ref[...])
    prl_ref[...] = pr[:, :H]
    prr_ref[...] = pr[:, H:]
    pcl_ref[...] = pc[:, :H]
    pcr_ref[...] = pc[:, H:]


def _k1(h_pad, w1r, w1c, be1):
    nb = NP // NBLK
    return pl.pallas_call(
        _k1_body,
        grid=(nb,),
        in_specs=[
            pl.BlockSpec((NBLK, D), lambda i: (i, 0)),
            pl.BlockSpec((D, D), lambda i: (0, 0)),
            pl.BlockSpec((D, D), lambda i: (0, 0)),
            pl.BlockSpec((1, D), lambda i: (0, 0)),
        ],
        out_specs=[pl.BlockSpec((NBLK, H), lambda i: (i, 0))] * 4,
        out_shape=[jax.ShapeDtypeStruct((NP, H), F32)] * 4,
    )(h_pad, w1r, w1c, be1)


def _k2_body(prl, prr, cp, pcl, pcr, cn, row_hbm, col_hbm,
             esl_hbm, esr_hbm, cd_hbm, *scr):
    ir = scr[0:SL]
    ic = scr[SL:2 * SL]
    bl = scr[2 * SL:3 * SL]
    br = scr[3 * SL:4 * SL]
    bc = scr[4 * SL:5 * SL]
    sa = scr[5 * SL:6 * SL]
    sb = scr[6 * SL:7 * SL]
    sw = scr[7 * SL:8 * SL]
    wid = lax.axis_index("s") * NC + lax.axis_index("c")
    base = wid * (EP // NW)
    nch = (EP // NW) // CH2
    G = nch // SL

    def A(c, p):
        st = base + c * CH2
        pltpu.sync_copy(row_hbm.at[pl.ds(st, CH2)], ir[p])
        pltpu.sync_copy(col_hbm.at[pl.ds(st, CH2)], ic[p])
        pltpu.async_copy(prl.at[ir[p]], bl[p], sa[p])
        pltpu.async_copy(prr.at[ir[p]], br[p], sa[p])
        pltpu.async_copy(cp.at[ir[p]], bc[p], sa[p])

    def Bp(c, p):
        pltpu.make_async_copy(prl.at[ir[p]], bl[p], sa[p]).wait()
        pltpu.make_async_copy(prr.at[ir[p]], br[p], sa[p]).wait()
        pltpu.make_async_copy(cp.at[ir[p]], bc[p], sa[p]).wait()
        pltpu.async_copy(pcl.at[ic[p]], bl[p], sb[p], add=True)
        pltpu.async_copy(pcr.at[ic[p]], br[p], sb[p], add=True)
        pltpu.async_copy(cn.at[ic[p]], bc[p], sb[p], add=True)

    def Cp(c, p):
        st = base + c * CH2
        pltpu.make_async_copy(pcl.at[ic[p]], bl[p], sb[p]).wait()
        pltpu.make_async_copy(pcr.at[ic[p]], br[p], sb[p]).wait()
        pltpu.make_async_copy(cn.at[ic[p]], bc[p], sb[p]).wait()
        pltpu.async_copy(bl[p], esl_hbm.at[pl.ds(st, CH2)], sw[p])
        pltpu.async_copy(br[p], esr_hbm.at[pl.ds(st, CH2)], sw[p])
        pltpu.async_copy(bc[p], cd_hbm.at[pl.ds(st, CH2)], sw[p])

    def Dp(c, p):
        st = base + c * CH2
        pltpu.make_async_copy(bl[p], esl_hbm.at[pl.ds(st, CH2)], sw[p]).wait()
        pltpu.make_async_copy(br[p], esr_hbm.at[pl.ds(st, CH2)], sw[p]).wait()
        pltpu.make_async_copy(bc[p], cd_hbm.at[pl.ds(st, CH2)], sw[p]).wait()

    A(0, 0)
    A(1, 1); Bp(0, 0)
    A(2, 2); Bp(1, 1); Cp(0, 0)
    A(3, 3); Bp(2, 2); Cp(1, 1)

    def group(g, carry):
        for off in range(SL):
            k = g * SL + off
            Dp(k - SL, off)
            A(k, off)
            Bp(k - 1, (off - 1) % SL)
            Cp(k - 2, (off - 2) % SL)
        return carry

    lax.fori_loop(1, G, group, 0)

    Dp(nch - 4, 0); Bp(nch - 1, 3); Cp(nch - 2, 2)
    Dp(nch - 3, 1); Cp(nch - 1, 3)
    Dp(nch - 2, 2)
    Dp(nch - 1, 3)


def _k2(prl, prr, cp, pcl, pcr, cn, row2, col2):
    mesh = plsc.VectorSubcoreMesh(core_axis_name="c", subcore_axis_name="s")
    scratch = ([pltpu.VMEM((CH2,), jnp.int32) for _ in range(2 * SL)]
               + [pltpu.VMEM((CH2, H), F32) for _ in range(2 * SL)]
               + [pltpu.VMEM((CH2, 16), F32) for _ in range(SL)]
               + [pltpu.SemaphoreType.DMA for _ in range(3 * SL)])
    fn = pl.kernel(
        _k2_body,
        out_type=[
            jax.ShapeDtypeStruct((EP, H), F32),
            jax.ShapeDtypeStruct((EP, H), F32),
            jax.ShapeDtypeStruct((EP, 16), F32),
        ],
        mesh=mesh,
        scratch_types=scratch,
        compiler_params=pltpu.CompilerParams(use_tc_tiling_on_sc=False),
    )
    return fn(prl, prr, cp, pcl, pcr, cn, row2, col2)


def _k3_body(esl_ref, esr_ref, cd_ref, w1l_ref, w1r_ref,
             we2a_ref, we2b_ref, be2_ref, wc1_ref, bc1_ref, wc2_ref,
             efl_ref, efr_ref, tr_ref):
    cd = cd_ref[...]
    rad = jnp.sum(cd * cd, axis=1, keepdims=True)
    tl = _silu(esl_ref[...] + rad * w1l_ref[...])
    tr_ = _silu(esr_ref[...] + rad * w1r_ref[...])
    m = _silu(_bdot(tl, we2a_ref[...]) + _bdot(tr_, we2b_ref[...])
              + be2_ref[...])
    efl_ref[...] = m[:, :H]
    efr_ref[...] = m[:, H:]
    s = _silu(_bdot(m, wc1_ref[...]) + bc1_ref[...])
    c = _bdot(s, wc2_ref[...])
    cnt1 = (lax.broadcasted_iota(jnp.int32, (1, 16), 1) == 3).astype(F32)
    tr_ref[...] = cd * c + cnt1


def _k3(esl, esr, cd16, w1radl, w1radr, we2a, we2b, be2, wc1, bc1, wc2):
    nb = EP // EBLK
    return pl.pallas_call(
        _k3_body,
        grid=(nb,),
        in_specs=[
            pl.BlockSpec((EBLK, H), lambda i: (i, 0)),
            pl.BlockSpec((EBLK, H), lambda i: (i, 0)),
            pl.BlockSpec((EBLK, 16), lambda i: (i, 0)),
            pl.BlockSpec((1, H), lambda i: (0, 0)),
            pl.BlockSpec((1, H), lambda i: (0, 0)),
            pl.BlockSpec((H, D), lambda i: (0, 0)),
            pl.BlockSpec((H, D), lambda i: (0, 0)),
            pl.BlockSpec((1, D), lambda i: (0, 0)),
            pl.BlockSpec((D, D), lambda i: (0, 0)),
            pl.BlockSpec((1, D), lambda i: (0, 0)),
            pl.BlockSpec((D, 1), lambda i: (0, 0)),
        ],
        out_specs=[
            pl.BlockSpec((EBLK, H), lambda i: (i, 0)),
            pl.BlockSpec((EBLK, H), lambda i: (i, 0)),
            pl.BlockSpec((EBLK, 16), lambda i: (i, 0)),
        ],
        out_shape=[
            jax.ShapeDtypeStruct((EP, H), F32),
            jax.ShapeDtypeStruct((EP, H), F32),
            jax.ShapeDtypeStruct((EP, 16), F32),
        ],
    )(esl, esr, cd16, w1radl, w1radr, we2a, we2b, be2, wc1, bc1, wc2)


def _k4_body(efl_hbm, efr_hbm, tr_hbm, row_hbm, zer_hbm, zer16_hbm,
             aggl_hbm, aggr_hbm, tc_hbm, *scr):
    iv = scr[0:SL]
    sx = scr[SL:2 * SL]
    bl = scr[2 * SL:3 * SL]
    br = scr[3 * SL:4 * SL]
    bt = scr[4 * SL:5 * SL]
    sr = scr[5 * SL:6 * SL]
    ss = scr[6 * SL:7 * SL]
    tabl = scr[7 * SL]
    tabr = scr[7 * SL + 1]
    tabt = scr[7 * SL + 2]
    cid = lax.axis_index("c")
    sid = lax.axis_index("s")
    sc_base = cid * HALF
    base = sid * (EP // NS)
    nch = (EP // NS) // CH4
    G = nch // SL
    pltpu.sync_copy(zer_hbm, tabl.at[pl.ds(sid * 320, 320)])
    pltpu.sync_copy(zer_hbm, tabr.at[pl.ds(sid * 320, 320)])
    pltpu.sync_copy(zer16_hbm, tabt.at[pl.ds(sid * 320, 320)])
    plsc.subcore_barrier()

    def A(c, p):
        st = base + c * CH4
        pltpu.sync_copy(row_hbm.at[pl.ds(st, CH4)], iv[p])
        pltpu.async_copy(efl_hbm.at[pl.ds(st, CH4)], bl[p], sr[p])
        pltpu.async_copy(efr_hbm.at[pl.ds(st, CH4)], br[p], sr[p])
        pltpu.async_copy(tr_hbm.at[pl.ds(st, CH4)], bt[p], sr[p])

    def Bp(c, p):
        pltpu.make_async_copy(efl_hbm.at[pl.ds(0, CH4)], bl[p], sr[p]).wait()
        pltpu.make_async_copy(efr_hbm.at[pl.ds(0, CH4)], br[p], sr[p]).wait()
        pltpu.make_async_copy(tr_hbm.at[pl.ds(0, CH4)], bt[p], sr[p]).wait()
        for j in range(CH4 // 16):
            sl = pl.ds(16 * j, 16)
            x = iv[p][sl] - sc_base
            ok = (x >= 0) & (x < HALF)
            sx[p][sl] = jnp.where(ok, x, DUMMY)
        pltpu.async_copy(bl[p], tabl.at[sx[p]], ss[p], add=True)
        pltpu.async_copy(br[p], tabr.at[sx[p]], ss[p], add=True)
        pltpu.async_copy(bt[p], tabt.at[sx[p]], ss[p], add=True)

    def Cp(c, p):
        pltpu.make_async_copy(bl[p], tabl.at[sx[p]], ss[p]).wait()
        pltpu.make_async_copy(br[p], tabr.at[sx[p]], ss[p]).wait()
        pltpu.make_async_copy(bt[p], tabt.at[sx[p]], ss[p]).wait()

    A(0, 0)
    A(1, 1); Bp(0, 0)
    A(2, 2); Bp(1, 1); Cp(0, 0)
    A(3, 3); Bp(2, 2); Cp(1, 1)

    def group(g, carry):
        for off in range(SL):
            k = g * SL + off
            A(k, off)
            Bp(k - 1, (off - 1) % SL)
            Cp(k - 2, (off - 2) % SL)
        return carry

    lax.fori_loop(1, G, group, 0)
    Bp(nch - 1, 3); Cp(nch - 2, 2)
    Cp(nch - 1, 3)

    plsc.subcore_barrier()
    pltpu.sync_copy(tabl.at[pl.ds(sid * 320, 320)],
                    aggl_hbm.at[pl.ds(sc_base + sid * 320, 320)])
    pltpu.sync_copy(tabr.at[pl.ds(sid * 320, 320)],
                    aggr_hbm.at[pl.ds(sc_base + sid * 320, 320)])
    pltpu.sync_copy(tabt.at[pl.ds(sid * 320, 320)],
                    tc_hbm.at[pl.ds(sc_base + sid * 320, 320)])


def _k4(efl, efr, tr16, row4, zer, zer16):
    mesh = plsc.VectorSubcoreMesh(core_axis_name="c", subcore_axis_name="s")
    scratch = ([pltpu.VMEM((CH4,), jnp.int32) for _ in range(2 * SL)]
               + [pltpu.VMEM((CH4, H), F32) for _ in range(2 * SL)]
               + [pltpu.VMEM((CH4, 16), F32) for _ in range(SL)]
               + [pltpu.SemaphoreType.DMA for _ in range(2 * SL)]
               + [pltpu.VMEM_SHARED((TROWS, H), F32),
                  pltpu.VMEM_SHARED((TROWS, H), F32),
                  pltpu.VMEM_SHARED((TROWS, 16), F32)])
    fn = pl.kernel(
        _k4_body,
        out_type=[
            jax.ShapeDtypeStruct((NP, H), F32),
            jax.ShapeDtypeStruct((NP, H), F32),
            jax.ShapeDtypeStruct((NP, 16), F32),
        ],
        mesh=mesh,
        scratch_types=scratch,
        compiler_params=pltpu.CompilerParams(use_tc_tiling_on_sc=False),
    )
    return fn(efl, efr, tr16, row4, zer, zer16)


def _k5_body(h_ref, aggl_ref, aggr_ref, tcb_ref, co_ref,
             wn1a_ref, wn1bl_ref, wn1br_ref, wn2_ref,
             bn1_ref, bn2_ref, h1_ref, cout_ref, stat_ref):
    pid = pl.program_id(0)
    h = h_ref[...]
    t = _silu(_bdot(h, wn1a_ref[...]) + _bdot(aggl_ref[...], wn1bl_ref[...])
              + _bdot(aggr_ref[...], wn1br_ref[...]) + bn1_ref[...])
    hn = _bdot(t, wn2_ref[...]) + bn2_ref[...]
    h1 = _elu(h + hn)
    rid = lax.broadcasted_iota(jnp.int32, (NBLK, 1), 0) + pid * NBLK
    h1 = jnp.where(rid < N0, h1, 0.0)
    h1_ref[...] = h1
    tcb = tcb_ref[...]
    cnt = jnp.clip(tcb[:, 3:4], 1.0, None)
    cout_ref[...] = co_ref[...] + tcb / cnt
    s1 = jnp.sum(h1, axis=0, keepdims=True)
    s2 = jnp.sum(h1 * h1, axis=0, keepdims=True)
    st = jnp.concatenate([s1, s2, jnp.zeros((6, D), F32)], axis=0)

    @pl.when(pid == 0)
    def _():
        stat_ref[...] = st

    @pl.when(pid != 0)
    def _():
        stat_ref[...] = stat_ref[...] + st


def _k5(h_pad, aggl, aggr, tctab, coord16, wn1a, wn1bl, wn1br, wn2, bn1, bn2):
    nb = NP // NBLK
    return pl.pallas_call(
        _k5_body,
        grid=(nb,),
        in_specs=[
            pl.BlockSpec((NBLK, D), lambda i: (i, 0)),
            pl.BlockSpec((NBLK, H), lambda i: (i, 0)),
            pl.BlockSpec((NBLK, H), lambda i: (i, 0)),
            pl.BlockSpec((NBLK, 16), lambda i: (i, 0)),
            pl.BlockSpec((NBLK, 16), lambda i: (i, 0)),
            pl.BlockSpec((D, D), lambda i: (0, 0)),
            pl.BlockSpec((H, D), lambda i: (0, 0)),
            pl.BlockSpec((H, D), lambda i: (0, 0)),
            pl.BlockSpec((D, D), lambda i: (0, 0)),
            pl.BlockSpec((1, D), lambda i: (0, 0)),
            pl.BlockSpec((1, D), lambda i: (0, 0)),
        ],
        out_specs=[
            pl.BlockSpec((NBLK, D), lambda i: (i, 0)),
            pl.BlockSpec((NBLK, 16), lambda i: (i, 0)),
            pl.BlockSpec((8, D), lambda i: (0, 0)),
        ],
        out_shape=[
            jax.ShapeDtypeStruct((NP, D), F32),
            jax.ShapeDtypeStruct((NP, 16), F32),
            jax.ShapeDtypeStruct((8, D), F32),
        ],
    )(h_pad, aggl, aggr, tctab, coord16, wn1a, wn1bl, wn1br, wn2, bn1, bn2)


def _k6_body(stat_ref, h1_ref, g_ref, b_ref, wm_ref, bm_ref, h1n_ref, x_ref):
    mu = stat_ref[0:1, :] * (1.0 / N0)
    ex2 = stat_ref[1:2, :] * (1.0 / N0)
    var = ex2 - mu * mu
    inv = lax.rsqrt(var + 1e-5)
    h1n = (h1_ref[...] - mu) * inv * g_ref[...] + b_ref[...]
    h1n_ref[...] = h1n
    x_ref[...] = _elu(_bdot(h1n, wm_ref[...]) + bm_ref[...])


def _k6(stat, h1, gamma, beta, wm, bm):
    nb = NP // NBLK
    return pl.pallas_call(
        _k6_body,
        grid=(nb,),
        in_specs=[
            pl.BlockSpec((8, D), lambda i: (0, 0)),
            pl.BlockSpec((NBLK, D), lambda i: (i, 0)),
            pl.BlockSpec((1, D), lambda i: (0, 0)),
            pl.BlockSpec((1, D), lambda i: (0, 0)),
            pl.BlockSpec((D, D), lambda i: (0, 0)),
            pl.BlockSpec((1, D), lambda i: (0, 0)),
        ],
        out_specs=[
            pl.BlockSpec((NBLK, D), lambda i: (i, 0)),
            pl.BlockSpec((NBLK, D), lambda i: (i, 0)),
        ],
        out_shape=[
            jax.ShapeDtypeStruct((NP, D), F32),
            jax.ShapeDtypeStruct((NP, D), F32),
        ],
    )(stat, h1, gamma, beta, wm, bm)


def _k7a_body(x_hbm, bnd_hbm, neg_hbm, zer_hbm,
              psum_hbm, pmax_hbm,
              xbuf, bndv, sumt, maxt):
    wid = lax.axis_index("s") * NC + lax.axis_index("c")
    base = wid * 320
    pltpu.sync_copy(zer_hbm.at[pl.ds(0, GT)], sumt)
    pltpu.sync_copy(neg_hbm, maxt)
    pltpu.sync_copy(bnd_hbm, bndv)
    pltpu.sync_copy(x_hbm.at[pl.ds(base, 320)], xbuf)

    bvals = [bndv[pl.ds(16 * k, 16)] for k in range(5)]
    NV = D // 16

    for g in range(B + 1):
        lo = bvals[g // 16][g % 16] - base
        hi = bvals[(g + 1) // 16][(g + 1) % 16] - base
        lo = jnp.minimum(jnp.maximum(lo, 0), 320)
        hi = jnp.minimum(jnp.maximum(hi, 0), 320)

        def rowf(r, carry):
            accs = carry[:NV]
            accm = carry[NV:2 * NV]
            vs = [xbuf[r, pl.ds(16 * j, 16)] for j in range(NV)]
            accs = tuple(a + v for a, v in zip(accs, vs))
            accm = tuple(jnp.maximum(a, v) for a, v in zip(accm, vs))
            return accs + accm

        init = (tuple(jnp.zeros((16,), F32) for _ in range(NV))
                + tuple(jnp.full((16,), -1e30, F32) for _ in range(NV)))
        res = lax.fori_loop(lo, hi, rowf, init)
        for j in range(NV):
            sumt[g, pl.ds(16 * j, 16)] = res[j]
            maxt[g, pl.ds(16 * j, 16)] = res[NV + j]

    pltpu.sync_copy(sumt, psum_hbm.at[pl.ds(wid * GT, GT)])
    pltpu.sync_copy(maxt, pmax_hbm.at[pl.ds(wid * GT, GT)])


def _k7a(x, bnd, neg, zer):
    mesh = plsc.VectorSubcoreMesh(core_axis_name="c", subcore_axis_name="s")
    fn = pl.kernel(
        _k7a_body,
        out_type=[
            jax.ShapeDtypeStruct((NW * GT, D), F32),
            jax.ShapeDtypeStruct((NW * GT, D), F32),
        ],
        mesh=mesh,
        scratch_types=[
            pltpu.VMEM((320, D), F32),
            pltpu.VMEM((80,), jnp.int32),
            pltpu.VMEM((GT, D), F32),
            pltpu.VMEM((GT, D), F32),
        ],
        compiler_params=pltpu.CompilerParams(use_tc_tiling_on_sc=False),
    )
    return fn(x, bnd, neg, zer)


def _k7b_body(ps_ref, pm_ref, bc_ref, cw_ref, cb_ref, pert_ref):
    xsum = jnp.sum(ps_ref[...], axis=0)
    xmax = jnp.max(pm_ref[...], axis=0)
    cnt = bc_ref[...]
    xmean = xsum / jnp.clip(cnt, 1.0, None)
    xmax = jnp.where(xmax < -8e29, 0.0, xmax)
    z = cw_ref[0, 0] * xmax + cw_ref[0, 1] * xmean + cb_ref[0, 0]
    pert_ref[...] = _elu(z)


def _k7b(psum3, pmax3, bcnt72, cw2d, cb2d):
    return pl.pallas_call(
        _k7b_body,
        grid=(1,),
        in_specs=[
            pl.BlockSpec((NW, GT, D), lambda i: (0, 0, 0)),
            pl.BlockSpec((NW, GT, D), lambda i: (0, 0, 0)),
            pl.BlockSpec((GT, 1), lambda i: (0, 0)),
            pl.BlockSpec(memory_space=pltpu.SMEM),
            pl.BlockSpec(memory_space=pltpu.SMEM),
        ],
        out_specs=[pl.BlockSpec((GT, D), lambda i: (0, 0))],
        out_shape=[jax.ShapeDtypeStruct((GT, D), F32)],
    )(psum3, pmax3, bcnt72, cw2d, cb2d)


def _k7c_body(h1n_ref, b_ref, pert_ref, out_ref):
    bb = b_ref[...]
    oh = (bb == lax.broadcasted_iota(jnp.int32, (1, B), 1)).astype(F32)
    add = _dot(oh, pert_ref[0:B, :])
    out_ref[...] = h1n_ref[...] + add


def _k7c(h1n, batch2d, pert):
    nb = NP // NBLK
    return pl.pallas_call(
        _k7c_body,
        grid=(nb,),
        in_specs=[
            pl.BlockSpec((NBLK, D), lambda i: (i, 0)),
            pl.BlockSpec((NBLK, 1), lambda i: (i, 0)),
            pl.BlockSpec((GT, D), lambda i: (0, 0)),
        ],
        out_specs=[pl.BlockSpec((NBLK, D), lambda i: (i, 0))],
        out_shape=[jax.ShapeDtypeStruct((NP, D), F32)],
    )(h1n, batch2d, pert)


def kernel(h, edge_index, coord, batch,
           W_e1, b_e1, W_e2, b_e2, W_n1, b_n1, W_n2, b_n2,
           W_c1, b_c1, W_c2, bn_gamma, bn_beta,
           W_m, b_m, conv_w, conv_b):
    row = edge_index[0]
    col = edge_index[1]
    padn = NP - N0
    pade = EP - E0

    h_pad = jnp.concatenate([h, jnp.zeros((padn, D), F32)], axis=0)
    coord16 = jnp.concatenate([coord, jnp.zeros((N0, 13), F32)], axis=1)
    coord16 = jnp.concatenate([coord16, jnp.zeros((padn, 16), F32)], axis=0)
    coordn16 = -coord16
    row2 = jnp.concatenate([row, jnp.zeros((pade,), jnp.int32)])
    col2 = jnp.concatenate([col, jnp.zeros((pade,), jnp.int32)])
    row4 = jnp.concatenate([row, jnp.full((pade,), NP, jnp.int32)])
    batch_pad = jnp.concatenate([batch, jnp.full((padn,), B, jnp.int32)])
    batch2d = batch_pad.reshape(NP, 1)
    bnd = jnp.searchsorted(batch_pad, jnp.arange(66, dtype=jnp.int32),
                           side='left').astype(jnp.int32)
    bcnt = (bnd[1:66] - bnd[0:65]).astype(F32)
    bcnt72 = jnp.concatenate([bcnt, jnp.zeros((7,), F32)]).reshape(GT, 1)
    bnd = jnp.concatenate([bnd, jnp.full((14,), NP, jnp.int32)])

    w1r = W_e1[:D]
    w1c = W_e1[D:2 * D]
    w1radl = W_e1[2 * D:2 * D + 1, :H]
    w1radr = W_e1[2 * D:2 * D + 1, H:]
    be1 = b_e1.reshape(1, D)
    we2a = W_e2[:H]
    we2b = W_e2[H:]
    be2 = b_e2.reshape(1, D)
    bc1 = b_c1.reshape(1, D)
    wn1a = W_n1[:D]
    wn1bl = W_n1[D:D + H]
    wn1br = W_n1[D + H:]
    bn1 = b_n1.reshape(1, D)
    bn2 = b_n2.reshape(1, D)
    gamma = bn_gamma.reshape(1, D)
    beta = bn_beta.reshape(1, D)
    bm = b_m.reshape(1, D)
    cw2d = conv_w.reshape(1, 2)
    cb2d = conv_b.reshape(1, 1)

    zer = jnp.zeros((320, H), F32)
    zer16 = jnp.zeros((320, 16), F32)
    zer256 = jnp.zeros((320, D), F32)
    neg = jnp.full((GT, D), -1e30, F32)

    w1r = w1r.astype(BF)
    w1c = w1c.astype(BF)
    we2a = we2a.astype(BF)
    we2b = we2b.astype(BF)
    wc1b = W_c1.astype(BF)
    wc2b = W_c2.astype(BF)
    wn1a = wn1a.astype(BF)
    wn1bl = wn1bl.astype(BF)
    wn1br = wn1br.astype(BF)
    wn2b = W_n2.astype(BF)
    wmb = W_m.astype(BF)

    prl, prr, pcl, pcr = _k1(h_pad, w1r, w1c, be1)
    esl, esr, cd16 = _k2(prl, prr, coord16, pcl, pcr, coordn16, row2, col2)
    efl, efr, tr16 = _k3(esl, esr, cd16, w1radl, w1radr,
                         we2a, we2b, be2, wc1b, bc1, wc2b)
    aggl, aggr, tctab = _k4(efl, efr, tr16, row4, zer, zer16)
    h1, cout16, stat = _k5(h_pad, aggl, aggr, tctab, coord16,
                           wn1a, wn1bl, wn1br, wn2b, bn1, bn2)
    h1n, x = _k6(stat, h1, gamma, beta, wmb, bm)
    psum, pmax = _k7a(x, bnd, neg, zer256)
    (pert,) = _k7b(psum.reshape(NW, GT, D), pmax.reshape(NW, GT, D),
                   bcnt72, cw2d, cb2d)
    (h_out,) = _k7c(h1n, batch2d, pert)

    return (h_out[:N0], cout16[:N0, :3])


# speedup vs baseline: 1.0249x; 1.0249x over previous
import jax
import jax.numpy as jnp
from jax import lax
from jax.experimental import pallas as pl
from jax.experimental.pallas import tpu as pltpu
from jax.experimental.pallas import tpu_sc as plsc

N0 = 10000
NP = 10240
E0 = 160000
EP = 163840
D = 256
H = 128
B = 64
NC = 2
NS = 16
NW = NC * NS
HALF = NP // NC
TROWS = HALF + 8
DUMMY = HALF
GT = 72
NBLK = 1024
EBLK = 512
CH2 = 64
CH4 = 32
SL = 4
F32 = jnp.float32


def _silu(x):
    return x * jax.nn.sigmoid(x)


def _elu(x):
    return jnp.where(x > 0, x, jnp.exp(jnp.minimum(x, 0.0)) - 1.0)


def _dot(a, b):
    return jnp.dot(a, b, preferred_element_type=F32)


BF = jnp.bfloat16


def _bdot(a, b):
    return jnp.dot(a.astype(BF), b, preferred_element_type=F32)


def _k1_body(h_ref, w1r_ref, w1c_ref, be1_ref,
             prl_ref, prr_ref, pcl_ref, pcr_ref):
    h = h_ref[...]
    pr = _dot(h, w1r_ref[...]) + be1_ref[...]
    pc = _dot(h, w1c_ref[...])
    prl_ref[...] = pr[:, :H]
    prr_ref[...] = pr[:, H:]
    pcl_ref[...] = pc[:, :H]
    pcr_ref[...] = pc[:, H:]


def _k1(h_pad, w1r, w1c, be1):
    nb = NP // NBLK
    return pl.pallas_call(
        _k1_body,
        grid=(nb,),
        in_specs=[
            pl.BlockSpec((NBLK, D), lambda i: (i, 0)),
            pl.BlockSpec((D, D), lambda i: (0, 0)),
            pl.BlockSpec((D, D), lambda i: (0, 0)),
            pl.BlockSpec((1, D), lambda i: (0, 0)),
        ],
        out_specs=[pl.BlockSpec((NBLK, H), lambda i: (i, 0))] * 4,
        out_shape=[jax.ShapeDtypeStruct((NP, H), F32)] * 4,
    )(h_pad, w1r, w1c, be1)


def _k2_body(prl, prr, cp, pcl, pcr, cn, row_hbm, col_hbm,
             esl_hbm, esr_hbm, cd_hbm, *scr):
    ir = scr[0:SL]
    ic = scr[SL:2 * SL]
    bl = scr[2 * SL:3 * SL]
    br = scr[3 * SL:4 * SL]
    bc = scr[4 * SL:5 * SL]
    sa = scr[5 * SL:6 * SL]
    sb = scr[6 * SL:7 * SL]
    sw = scr[7 * SL:8 * SL]
    wid = lax.axis_index("s") * NC + lax.axis_index("c")
    base = wid * (EP // NW)
    nch = (EP // NW) // CH2
    G = nch // SL

    def A(c, p):
        st = base + c * CH2
        pltpu.sync_copy(row_hbm.at[pl.ds(st, CH2)], ir[p])
        pltpu.sync_copy(col_hbm.at[pl.ds(st, CH2)], ic[p])
        pltpu.async_copy(prl.at[ir[p]], bl[p], sa[p])
        pltpu.async_copy(prr.at[ir[p]], br[p], sa[p])
        pltpu.async_copy(cp.at[ir[p]], bc[p], sa[p])

    def Bp(c, p):
        pltpu.make_async_copy(prl.at[ir[p]], bl[p], sa[p]).wait()
        pltpu.make_async_copy(prr.at[ir[p]], br[p], sa[p]).wait()
        pltpu.make_async_copy(cp.at[ir[p]], bc[p], sa[p]).wait()
        pltpu.async_copy(pcl.at[ic[p]], bl[p], sb[p], add=True)
        pltpu.async_copy(pcr.at[ic[p]], br[p], sb[p], add=True)
        pltpu.async_copy(cn.at[ic[p]], bc[p], sb[p], add=True)

    def Cp(c, p):
        st = base + c * CH2
        pltpu.make_async_copy(pcl.at[ic[p]], bl[p], sb[p]).wait()
        pltpu.make_async_copy(pcr.at[ic[p]], br[p], sb[p]).wait()
        pltpu.make_async_copy(cn.at[ic[p]], bc[p], sb[p]).wait()
        pltpu.async_copy(bl[p], esl_hbm.at[pl.ds(st, CH2)], sw[p])
        pltpu.async_copy(br[p], esr_hbm.at[pl.ds(st, CH2)], sw[p])
        pltpu.async_copy(bc[p], cd_hbm.at[pl.ds(st, CH2)], sw[p])

    def Dp(c, p):
        st = base + c * CH2
        pltpu.make_async_copy(bl[p], esl_hbm.at[pl.ds(st, CH2)], sw[p]).wait()
        pltpu.make_async_copy(br[p], esr_hbm.at[pl.ds(st, CH2)], sw[p]).wait()
        pltpu.make_async_copy(bc[p], cd_hbm.at[pl.ds(st, CH2)], sw[p]).wait()

    A(0, 0)
    A(1, 1); Bp(0, 0)
    A(2, 2); Bp(1, 1); Cp(0, 0)
    A(3, 3); Bp(2, 2); Cp(1, 1)

    def group(g, carry):
        for off in range(SL):
            k = g * SL + off
            Dp(k - SL, off)
            A(k, off)
            Bp(k - 1, (off - 1) % SL)
            Cp(k - 2, (off - 2) % SL)
        return carry

    lax.fori_loop(1, G, group, 0)

    Dp(nch - 4, 0); Bp(nch - 1, 3); Cp(nch - 2, 2)
    Dp(nch - 3, 1); Cp(nch - 1, 3)
    Dp(nch - 2, 2)
    Dp(nch - 1, 3)


def _k2(prl, prr, cp, pcl, pcr, cn, row2, col2):
    mesh = plsc.VectorSubcoreMesh(core_axis_name="c", subcore_axis_name="s")
    scratch = ([pltpu.VMEM((CH2,), jnp.int32) for _ in range(2 * SL)]
               + [pltpu.VMEM((CH2, H), F32) for _ in range(2 * SL)]
               + [pltpu.VMEM((CH2, 16), F32) for _ in range(SL)]
               + [pltpu.SemaphoreType.DMA for _ in range(3 * SL)])
    fn = pl.kernel(
        _k2_body,
        out_type=[
            jax.ShapeDtypeStruct((EP, H), F32),
            jax.ShapeDtypeStruct((EP, H), F32),
            jax.ShapeDtypeStruct((EP, 16), F32),
        ],
        mesh=mesh,
        scratch_types=scratch,
        compiler_params=pltpu.CompilerParams(use_tc_tiling_on_sc=False),
    )
    return fn(prl, prr, cp, pcl, pcr, cn, row2, col2)


def _k3_body(esl_ref, esr_ref, cd_ref, w1l_ref, w1r_ref,
             we2a_ref, we2b_ref, be2_ref, wc1_ref, bc1_ref, wc2_ref,
             efl_ref, efr_ref, tr_ref):
    cd = cd_ref[...]
    rad = jnp.sum(cd * cd, axis=1, keepdims=True)
    tl = _silu(esl_ref[...] + rad * w1l_ref[...])
    tr_ = _silu(esr_ref[...] + rad * w1r_ref[...])
    m = _silu(_dot(tl, we2a_ref[...]) + _dot(tr_, we2b_ref[...])
              + be2_ref[...])
    efl_ref[...] = m[:, :H]
    efr_ref[...] = m[:, H:]
    s = _silu(_dot(m, wc1_ref[...]) + bc1_ref[...])
    c = _dot(s, wc2_ref[...])
    cnt1 = (lax.broadcasted_iota(jnp.int32, (1, 16), 1) == 3).astype(F32)
    tr_ref[...] = cd * c + cnt1


def _k3(esl, esr, cd16, w1radl, w1radr, we2a, we2b, be2, wc1, bc1, wc2):
    nb = EP // EBLK
    return pl.pallas_call(
        _k3_body,
        grid=(nb,),
        in_specs=[
            pl.BlockSpec((EBLK, H), lambda i: (i, 0)),
            pl.BlockSpec((EBLK, H), lambda i: (i, 0)),
            pl.BlockSpec((EBLK, 16), lambda i: (i, 0)),
            pl.BlockSpec((1, H), lambda i: (0, 0)),
            pl.BlockSpec((1, H), lambda i: (0, 0)),
            pl.BlockSpec((H, D), lambda i: (0, 0)),
            pl.BlockSpec((H, D), lambda i: (0, 0)),
            pl.BlockSpec((1, D), lambda i: (0, 0)),
            pl.BlockSpec((D, D), lambda i: (0, 0)),
            pl.BlockSpec((1, D), lambda i: (0, 0)),
            pl.BlockSpec((D, 1), lambda i: (0, 0)),
        ],
        out_specs=[
            pl.BlockSpec((EBLK, H), lambda i: (i, 0)),
            pl.BlockSpec((EBLK, H), lambda i: (i, 0)),
            pl.BlockSpec((EBLK, 16), lambda i: (i, 0)),
        ],
        out_shape=[
            jax.ShapeDtypeStruct((EP, H), F32),
            jax.ShapeDtypeStruct((EP, H), F32),
            jax.ShapeDtypeStruct((EP, 16), F32),
        ],
    )(esl, esr, cd16, w1radl, w1radr, we2a, we2b, be2, wc1, bc1, wc2)


def _k4_body(efl_hbm, efr_hbm, tr_hbm, row_hbm, zer_hbm, zer16_hbm,
             aggl_hbm, aggr_hbm, tc_hbm, *scr):
    iv = scr[0:SL]
    sx = scr[SL:2 * SL]
    bl = scr[2 * SL:3 * SL]
    br = scr[3 * SL:4 * SL]
    bt = scr[4 * SL:5 * SL]
    sr = scr[5 * SL:6 * SL]
    ss = scr[6 * SL:7 * SL]
    tabl = scr[7 * SL]
    tabr = scr[7 * SL + 1]
    tabt = scr[7 * SL + 2]
    cid = lax.axis_index("c")
    sid = lax.axis_index("s")
    sc_base = cid * HALF
    base = sid * (EP // NS)
    nch = (EP // NS) // CH4
    G = nch // SL
    pltpu.sync_copy(zer_hbm, tabl.at[pl.ds(sid * 320, 320)])
    pltpu.sync_copy(zer_hbm, tabr.at[pl.ds(sid * 320, 320)])
    pltpu.sync_copy(zer16_hbm, tabt.at[pl.ds(sid * 320, 320)])
    plsc.subcore_barrier()

    def A(c, p):
        st = base + c * CH4
        pltpu.sync_copy(row_hbm.at[pl.ds(st, CH4)], iv[p])
        pltpu.async_copy(efl_hbm.at[pl.ds(st, CH4)], bl[p], sr[p])
        pltpu.async_copy(efr_hbm.at[pl.ds(st, CH4)], br[p], sr[p])
        pltpu.async_copy(tr_hbm.at[pl.ds(st, CH4)], bt[p], sr[p])

    def Bp(c, p):
        pltpu.make_async_copy(efl_hbm.at[pl.ds(0, CH4)], bl[p], sr[p]).wait()
        pltpu.make_async_copy(efr_hbm.at[pl.ds(0, CH4)], br[p], sr[p]).wait()
        pltpu.make_async_copy(tr_hbm.at[pl.ds(0, CH4)], bt[p], sr[p]).wait()
        for j in range(CH4 // 16):
            sl = pl.ds(16 * j, 16)
            x = iv[p][sl] - sc_base
            ok = (x >= 0) & (x < HALF)
            sx[p][sl] = jnp.where(ok, x, DUMMY)
        pltpu.async_copy(bl[p], tabl.at[sx[p]], ss[p], add=True)
        pltpu.async_copy(br[p], tabr.at[sx[p]], ss[p], add=True)
        pltpu.async_copy(bt[p], tabt.at[sx[p]], ss[p], add=True)

    def Cp(c, p):
        pltpu.make_async_copy(bl[p], tabl.at[sx[p]], ss[p]).wait()
        pltpu.make_async_copy(br[p], tabr.at[sx[p]], ss[p]).wait()
        pltpu.make_async_copy(bt[p], tabt.at[sx[p]], ss[p]).wait()

    A(0, 0)
    A(1, 1); Bp(0, 0)
    A(2, 2); Bp(1, 1); Cp(0, 0)
    A(3, 3); Bp(2, 2); Cp(1, 1)

    def group(g, carry):
        for off in range(SL):
            k = g * SL + off
            A(k, off)
            Bp(k - 1, (off - 1) % SL)
            Cp(k - 2, (off - 2) % SL)
        return carry

    lax.fori_loop(1, G, group, 0)
    Bp(nch - 1, 3); Cp(nch - 2, 2)
    Cp(nch - 1, 3)

    plsc.subcore_barrier()
    pltpu.sync_copy(tabl.at[pl.ds(sid * 320, 320)],
                    aggl_hbm.at[pl.ds(sc_base + sid * 320, 320)])
    pltpu.sync_copy(tabr.at[pl.ds(sid * 320, 320)],
                    aggr_hbm.at[pl.ds(sc_base + sid * 320, 320)])
    pltpu.sync_copy(tabt.at[pl.ds(sid * 320, 320)],
                    tc_hbm.at[pl.ds(sc_base + sid * 320, 320)])


def _k4(efl, efr, tr16, row4, zer, zer16):
    mesh = plsc.VectorSubcoreMesh(core_axis_name="c", subcore_axis_name="s")
    scratch = ([pltpu.VMEM((CH4,), jnp.int32) for _ in range(2 * SL)]
               + [pltpu.VMEM((CH4, H), F32) for _ in range(2 * SL)]
               + [pltpu.VMEM((CH4, 16), F32) for _ in range(SL)]
               + [pltpu.SemaphoreType.DMA for _ in range(2 * SL)]
               + [pltpu.VMEM_SHARED((TROWS, H), F32),
                  pltpu.VMEM_SHARED((TROWS, H), F32),
                  pltpu.VMEM_SHARED((TROWS, 16), F32)])
    fn = pl.kernel(
        _k4_body,
        out_type=[
            jax.ShapeDtypeStruct((NP, H), F32),
            jax.ShapeDtypeStruct((NP, H), F32),
            jax.ShapeDtypeStruct((NP, 16), F32),
        ],
        mesh=mesh,
        scratch_types=scratch,
        compiler_params=pltpu.CompilerParams(use_tc_tiling_on_sc=False),
    )
    return fn(efl, efr, tr16, row4, zer, zer16)


def _k5_body(h_ref, aggl_ref, aggr_ref, tcb_ref, co_ref,
             wn1a_ref, wn1bl_ref, wn1br_ref, wn2_ref,
             bn1_ref, bn2_ref, h1_ref, cout_ref, stat_ref):
    pid = pl.program_id(0)
    h = h_ref[...]
    t = _silu(_dot(h, wn1a_ref[...]) + _dot(aggl_ref[...], wn1bl_ref[...])
              + _dot(aggr_ref[...], wn1br_ref[...]) + bn1_ref[...])
    hn = _dot(t, wn2_ref[...]) + bn2_ref[...]
    h1 = _elu(h + hn)
    rid = lax.broadcasted_iota(jnp.int32, (NBLK, 1), 0) + pid * NBLK
    h1 = jnp.where(rid < N0, h1, 0.0)
    h1_ref[...] = h1
    tcb = tcb_ref[...]
    cnt = jnp.clip(tcb[:, 3:4], 1.0, None)
    cout_ref[...] = co_ref[...] + tcb / cnt
    s1 = jnp.sum(h1, axis=0, keepdims=True)
    s2 = jnp.sum(h1 * h1, axis=0, keepdims=True)
    st = jnp.concatenate([s1, s2, jnp.zeros((6, D), F32)], axis=0)

    @pl.when(pid == 0)
    def _():
        stat_ref[...] = st

    @pl.when(pid != 0)
    def _():
        stat_ref[...] = stat_ref[...] + st


def _k5(h_pad, aggl, aggr, tctab, coord16, wn1a, wn1bl, wn1br, wn2, bn1, bn2):
    nb = NP // NBLK
    return pl.pallas_call(
        _k5_body,
        grid=(nb,),
        in_specs=[
            pl.BlockSpec((NBLK, D), lambda i: (i, 0)),
            pl.BlockSpec((NBLK, H), lambda i: (i, 0)),
            pl.BlockSpec((NBLK, H), lambda i: (i, 0)),
            pl.BlockSpec((NBLK, 16), lambda i: (i, 0)),
            pl.BlockSpec((NBLK, 16), lambda i: (i, 0)),
            pl.BlockSpec((D, D), lambda i: (0, 0)),
            pl.BlockSpec((H, D), lambda i: (0, 0)),
            pl.BlockSpec((H, D), lambda i: (0, 0)),
            pl.BlockSpec((D, D), lambda i: (0, 0)),
            pl.BlockSpec((1, D), lambda i: (0, 0)),
            pl.BlockSpec((1, D), lambda i: (0, 0)),
        ],
        out_specs=[
            pl.BlockSpec((NBLK, D), lambda i: (i, 0)),
            pl.BlockSpec((NBLK, 16), lambda i: (i, 0)),
            pl.BlockSpec((8, D), lambda i: (0, 0)),
        ],
        out_shape=[
            jax.ShapeDtypeStruct((NP, D), F32),
            jax.ShapeDtypeStruct((NP, 16), F32),
            jax.ShapeDtypeStruct((8, D), F32),
        ],
    )(h_pad, aggl, aggr, tctab, coord16, wn1a, wn1bl, wn1br, wn2, bn1, bn2)


def _k6_body(stat_ref, h1_ref, g_ref, b_ref, wm_ref, bm_ref, h1n_ref, x_ref):
    mu = stat_ref[0:1, :] * (1.0 / N0)
    ex2 = stat_ref[1:2, :] * (1.0 / N0)
    var = ex2 - mu * mu
    inv = lax.rsqrt(var + 1e-5)
    h1n = (h1_ref[...] - mu) * inv * g_ref[...] + b_ref[...]
    h1n_ref[...] = h1n
    x_ref[...] = _elu(_dot(h1n, wm_ref[...]) + bm_ref[...])


def _k6(stat, h1, gamma, beta, wm, bm):
    nb = NP // NBLK
    return pl.pallas_call(
        _k6_body,
        grid=(nb,),
        in_specs=[
            pl.BlockSpec((8, D), lambda i: (0, 0)),
            pl.BlockSpec((NBLK, D), lambda i: (i, 0)),
            pl.BlockSpec((1, D), lambda i: (0, 0)),
            pl.BlockSpec((1, D), lambda i: (0, 0)),
            pl.BlockSpec((D, D), lambda i: (0, 0)),
            pl.BlockSpec((1, D), lambda i: (0, 0)),
        ],
        out_specs=[
            pl.BlockSpec((NBLK, D), lambda i: (i, 0)),
            pl.BlockSpec((NBLK, D), lambda i: (i, 0)),
        ],
        out_shape=[
            jax.ShapeDtypeStruct((NP, D), F32),
            jax.ShapeDtypeStruct((NP, D), F32),
        ],
    )(stat, h1, gamma, beta, wm, bm)


def _k7a_body(x_hbm, bnd_hbm, neg_hbm, zer_hbm,
              psum_hbm, pmax_hbm,
              xbuf, bndv, sumt, maxt):
    wid = lax.axis_index("s") * NC + lax.axis_index("c")
    base = wid * 320
    pltpu.sync_copy(zer_hbm.at[pl.ds(0, GT)], sumt)
    pltpu.sync_copy(neg_hbm, maxt)
    pltpu.sync_copy(bnd_hbm, bndv)
    pltpu.sync_copy(x_hbm.at[pl.ds(base, 320)], xbuf)

    bvals = [bndv[pl.ds(16 * k, 16)] for k in range(5)]
    NV = D // 16

    for g in range(B + 1):
        lo = bvals[g // 16][g % 16] - base
        hi = bvals[(g + 1) // 16][(g + 1) % 16] - base
        lo = jnp.minimum(jnp.maximum(lo, 0), 320)
        hi = jnp.minimum(jnp.maximum(hi, 0), 320)

        def rowf(r, carry):
            accs = carry[:NV]
            accm = carry[NV:2 * NV]
            vs = [xbuf[r, pl.ds(16 * j, 16)] for j in range(NV)]
            accs = tuple(a + v for a, v in zip(accs, vs))
            accm = tuple(jnp.maximum(a, v) for a, v in zip(accm, vs))
            return accs + accm

        init = (tuple(jnp.zeros((16,), F32) for _ in range(NV))
                + tuple(jnp.full((16,), -1e30, F32) for _ in range(NV)))
        res = lax.fori_loop(lo, hi, rowf, init)
        for j in range(NV):
            sumt[g, pl.ds(16 * j, 16)] = res[j]
            maxt[g, pl.ds(16 * j, 16)] = res[NV + j]

    pltpu.sync_copy(sumt, psum_hbm.at[pl.ds(wid * GT, GT)])
    pltpu.sync_copy(maxt, pmax_hbm.at[pl.ds(wid * GT, GT)])


def _k7a(x, bnd, neg, zer):
    mesh = plsc.VectorSubcoreMesh(core_axis_name="c", subcore_axis_name="s")
    fn = pl.kernel(
        _k7a_body,
        out_type=[
            jax.ShapeDtypeStruct((NW * GT, D), F32),
            jax.ShapeDtypeStruct((NW * GT, D), F32),
        ],
        mesh=mesh,
        scratch_types=[
            pltpu.VMEM((320, D), F32),
            pltpu.VMEM((80,), jnp.int32),
            pltpu.VMEM((GT, D), F32),
            pltpu.VMEM((GT, D), F32),
        ],
        compiler_params=pltpu.CompilerParams(use_tc_tiling_on_sc=False),
    )
    return fn(x, bnd, neg, zer)


def _k7b_body(ps_ref, pm_ref, bc_ref, cw_ref, cb_ref, pert_ref):
    xsum = jnp.sum(ps_ref[...], axis=0)
    xmax = jnp.max(pm_ref[...], axis=0)
    cnt = bc_ref[...]
    xmean = xsum / jnp.clip(cnt, 1.0, None)
    xmax = jnp.where(xmax < -8e29, 0.0, xmax)
    z = cw_ref[0, 0] * xmax + cw_ref[0, 1] * xmean + cb_ref[0, 0]
    pert_ref[...] = _elu(z)


def _k7b(psum3, pmax3, bcnt72, cw2d, cb2d):
    return pl.pallas_call(
        _k7b_body,
        grid=(1,),
        in_specs=[
            pl.BlockSpec((NW, GT, D), lambda i: (0, 0, 0)),
            pl.BlockSpec((NW, GT, D), lambda i: (0, 0, 0)),
            pl.BlockSpec((GT, 1), lambda i: (0, 0)),
            pl.BlockSpec(memory_space=pltpu.SMEM),
            pl.BlockSpec(memory_space=pltpu.SMEM),
        ],
        out_specs=[pl.BlockSpec((GT, D), lambda i: (0, 0))],
        out_shape=[jax.ShapeDtypeStruct((GT, D), F32)],
    )(psum3, pmax3, bcnt72, cw2d, cb2d)


def _k7c_body(h1n_ref, b_ref, pert_ref, out_ref):
    bb = b_ref[...]
    oh = (bb == lax.broadcasted_iota(jnp.int32, (1, B), 1)).astype(F32)
    add = _dot(oh, pert_ref[0:B, :])
    out_ref[...] = h1n_ref[...] + add


def _k7c(h1n, batch2d, pert):
    nb = NP // NBLK
    return pl.pallas_call(
        _k7c_body,
        grid=(nb,),
        in_specs=[
            pl.BlockSpec((NBLK, D), lambda i: (i, 0)),
            pl.BlockSpec((NBLK, 1), lambda i: (i, 0)),
            pl.BlockSpec((GT, D), lambda i: (0, 0)),
        ],
        out_specs=[pl.BlockSpec((NBLK, D), lambda i: (i, 0))],
        out_shape=[jax.ShapeDtypeStruct((NP, D), F32)],
    )(h1n, batch2d, pert)


def kernel(h, edge_index, coord, batch,
           W_e1, b_e1, W_e2, b_e2, W_n1, b_n1, W_n2, b_n2,
           W_c1, b_c1, W_c2, bn_gamma, bn_beta,
           W_m, b_m, conv_w, conv_b):
    row = edge_index[0]
    col = edge_index[1]
    padn = NP - N0
    pade = EP - E0

    h_pad = jnp.concatenate([h, jnp.zeros((padn, D), F32)], axis=0)
    coord16 = jnp.concatenate([coord, jnp.zeros((N0, 13), F32)], axis=1)
    coord16 = jnp.concatenate([coord16, jnp.zeros((padn, 16), F32)], axis=0)
    coordn16 = -coord16
    row2 = jnp.concatenate([row, jnp.zeros((pade,), jnp.int32)])
    col2 = jnp.concatenate([col, jnp.zeros((pade,), jnp.int32)])
    row4 = jnp.concatenate([row, jnp.full((pade,), NP, jnp.int32)])
    batch_pad = jnp.concatenate([batch, jnp.full((padn,), B, jnp.int32)])
    batch2d = batch_pad.reshape(NP, 1)
    bnd = jnp.searchsorted(batch_pad, jnp.arange(66, dtype=jnp.int32),
                           side='left').astype(jnp.int32)
    bcnt = (bnd[1:66] - bnd[0:65]).astype(F32)
    bcnt72 = jnp.concatenate([bcnt, jnp.zeros((7,), F32)]).reshape(GT, 1)
    bnd = jnp.concatenate([bnd, jnp.full((14,), NP, jnp.int32)])

    w1r = W_e1[:D]
    w1c = W_e1[D:2 * D]
    w1radl = W_e1[2 * D:2 * D + 1, :H]
    w1radr = W_e1[2 * D:2 * D + 1, H:]
    be1 = b_e1.reshape(1, D)
    we2a = W_e2[:H]
    we2b = W_e2[H:]
    be2 = b_e2.reshape(1, D)
    bc1 = b_c1.reshape(1, D)
    wn1a = W_n1[:D]
    wn1bl = W_n1[D:D + H]
    wn1br = W_n1[D + H:]
    bn1 = b_n1.reshape(1, D)
    bn2 = b_n2.reshape(1, D)
    gamma = bn_gamma.reshape(1, D)
    beta = bn_beta.reshape(1, D)
    bm = b_m.reshape(1, D)
    cw2d = conv_w.reshape(1, 2)
    cb2d = conv_b.reshape(1, 1)

    zer = jnp.zeros((320, H), F32)
    zer16 = jnp.zeros((320, 16), F32)
    zer256 = jnp.zeros((320, D), F32)
    neg = jnp.full((GT, D), -1e30, F32)

    prl, prr, pcl, pcr = _k1(h_pad, w1r, w1c, be1)
    esl, esr, cd16 = _k2(prl, prr, coord16, pcl, pcr, coordn16, row2, col2)
    efl, efr, tr16 = _k3(esl, esr, cd16, w1radl, w1radr,
                         we2a, we2b, be2, W_c1, bc1, W_c2)
    aggl, aggr, tctab = _k4(efl, efr, tr16, row4, zer, zer16)
    h1, cout16, stat = _k5(h_pad, aggl, aggr, tctab, coord16,
                           wn1a, wn1bl, wn1br, W_n2, bn1, bn2)
    h1n, x = _k6(stat, h1, gamma, beta, W_m, bm)
    psum, pmax = _k7a(x, bnd, neg, zer256)
    (pert,) = _k7b(psum.reshape(NW, GT, D), pmax.reshape(NW, GT, D),
                   bcnt72, cw2d, cb2d)
    (h_out,) = _k7c(h1n, batch2d, pert)

    return (h_out[:N0], cout16[:N0, :3])
